# baseline (speedup 1.0000x reference)
import dataclasses
import functools

import jax
import jax.numpy as jnp
from jax import lax
from jax.experimental import pallas as pl
from jax.experimental.pallas import tpu as pltpu
from jax.experimental.pallas import tpu_sc as plsc

_NUM_ENT = 1000000
_DIM = 32
_NUM_REL = 5
_BATCH = 16384
_NIDX = 2 * _BATCH
_PACK = 128 // _DIM

_NC = 2
_NS = 16
_NW = _NC * _NS
_BPW = _NIDX // _NW
_G = 16


def _sc_gather_rows(table3, c_idx, q_idx):
    mesh = plsc.VectorSubcoreMesh(core_axis_name="c", subcore_axis_name="s")
    cp = pltpu.CompilerParams()
    if "needs_layout_passes" in pltpu.CompilerParams.__dataclass_fields__:
        cp = dataclasses.replace(cp, needs_layout_passes=False)

    @functools.partial(
        pl.kernel,
        mesh=mesh,
        compiler_params=cp,
        out_type=jax.ShapeDtypeStruct((_NIDX // _PACK, 1, 128), jnp.float32),
        scratch_types=[
            pltpu.VMEM((_BPW,), jnp.int32),
            pltpu.VMEM((_BPW,), jnp.int32),
            pltpu.VMEM((_BPW // _PACK, 1, 128), jnp.float32),
            pltpu.SemaphoreType.DMA,
        ],
    )
    def k(tbl2_hbm, c_hbm, q_hbm, out_hbm, cv, qv, sel_v, sem):
        tbl_hbm = tbl2_hbm.reshape(_NUM_ENT // 8, 8, _DIM)
        wid = lax.axis_index("s") * _NC + lax.axis_index("c")
        base = wid * _BPW
        out_base = wid * (_BPW // _PACK)
        pltpu.sync_copy(c_hbm.at[pl.ds(base, _BPW)], cv)
        pltpu.sync_copy(q_hbm.at[pl.ds(base, _BPW)], qv)

        @pl.loop(0, _BPW // _G)
        def _grp(g):
            c16 = cv[pl.ds(g * _G, _G)]
            q16 = qv[pl.ds(g * _G, _G)]
            for j in range(_G):
                dr = g * (_G // _PACK) + j // _PACK
                lane = (j % _PACK) * _DIM
                pltpu.async_copy(
                    tbl_hbm.at[c16[j], q16[j], :],
                    sel_v.at[dr, 0, pl.ds(lane, _DIM)],
                    sem,
                )

        pltpu.make_async_copy(
            out_hbm.at[pl.ds(out_base, _BPW // _PACK)], sel_v, sem
        ).wait()
        pltpu.sync_copy(sel_v, out_hbm.at[pl.ds(out_base, _BPW // _PACK)])

    return k(table3, c_idx, q_idx)


def _tc_body(ht_ref, relb_ref, gamma_ref, beta_ref, rel_ref,
             preds_ref, loss_ref):
    nrows = _BATCH // _PACK
    h = ht_ref[0:nrows, :]
    t = ht_ref[nrows:2 * nrows, :]

    jj = lax.broadcasted_iota(jnp.int32, (128, 128), 0)
    kk = lax.broadcasted_iota(jnp.int32, (128, 128), 1)
    fold = jnp.where((jj % _DIM) == (kk % _DIM), 1.0 / _BATCH, 0.0)

    def batchnorm(x):
        m = lax.dot(jnp.sum(x, axis=0, keepdims=True), fold,
                    preferred_element_type=jnp.float32)
        xc = x - m
        v = lax.dot(jnp.sum(xc * xc, axis=0, keepdims=True), fold,
                    preferred_element_type=jnp.float32)
        scale = gamma_ref[...] * lax.rsqrt(v + 1e-5)
        return xc * scale + beta_ref[...]

    hn = batchnorm(h)
    tn = batchnorm(t)

    rel_id = relb_ref[...] % _NUM_REL
    r = jnp.zeros((nrows, 128), jnp.float32)
    for q in range(_NUM_REL):
        r = r + jnp.where(rel_id == q, rel_ref[q:q + 1, :], 0.0)

    p = hn * tn * r
    sj = lax.broadcasted_iota(jnp.int32, (128, _PACK), 0)
    sq = lax.broadcasted_iota(jnp.int32, (128, _PACK), 1)
    seg = jnp.where((sj // _DIM) == sq, 1.0, 0.0)
    score = lax.dot(p, seg, preferred_element_type=jnp.float32)
    preds = jax.nn.sigmoid(score)
    preds_ref[...] = preds
    loss = -jnp.sum(jnp.log(preds + 1e-10)) / _BATCH
    loss_ref[...] = jnp.broadcast_to(loss, (1, 1))


def _tc_decode(ht4, relb4, gamma128, beta128, rel_tab):
    nrows = _BATCH // _PACK
    return pl.pallas_call(
        _tc_body,
        out_shape=[
            jax.ShapeDtypeStruct((nrows, _PACK), jnp.float32),
            jax.ShapeDtypeStruct((1, 1), jnp.float32),
        ],
    )(ht4, relb4, gamma128, beta128, rel_tab)


def kernel(pos_edges, W, gamma, beta, rel_emb):
    idx = jnp.concatenate([pos_edges[:, 0], pos_edges[:, 2]]).astype(jnp.int32)
    c_idx = idx // 8
    q_idx = idx % 8
    ht4 = _sc_gather_rows(W, c_idx, q_idx).reshape(_NIDX // _PACK, 128)
    relb4 = jnp.broadcast_to(
        pos_edges[:, 1][:, None].astype(jnp.int32), (_BATCH, _DIM)
    ).reshape(_BATCH // _PACK, 128)
    gamma128 = jnp.tile(gamma.reshape(1, _DIM), (1, _PACK))
    beta128 = jnp.tile(beta.reshape(1, _DIM), (1, _PACK))
    rel_tab = jnp.tile(rel_emb, (1, _PACK))
    preds4, loss11 = _tc_decode(ht4, relb4, gamma128, beta128, rel_tab)
    return loss11[0, 0], preds4.reshape(_BATCH)


# speedup vs baseline: 1.5229x; 1.5229x over previous
import dataclasses
import functools

import jax
import jax.numpy as jnp
from jax import lax
from jax.experimental import pallas as pl
from jax.experimental.pallas import tpu as pltpu
from jax.experimental.pallas import tpu_sc as plsc

_NUM_ENT = 1000000
_DIM = 32
_NUM_REL = 5
_BATCH = 16384
_NIDX = 2 * _BATCH
_PACK = 128 // _DIM

_NC = 2
_NS = 16
_NW = _NC * _NS
_BPW = _NIDX // _NW
_G = 16


def _sc_gather_rows(table3, idx):
    mesh = plsc.VectorSubcoreMesh(core_axis_name="c", subcore_axis_name="s")
    cp = pltpu.CompilerParams()
    if "needs_layout_passes" in pltpu.CompilerParams.__dataclass_fields__:
        cp = dataclasses.replace(cp, needs_layout_passes=False)

    @functools.partial(
        pl.kernel,
        mesh=mesh,
        compiler_params=cp,
        out_type=jax.ShapeDtypeStruct((_NIDX // _PACK, 1, 128), jnp.float32),
        scratch_types=[
            pltpu.VMEM((_BPW,), jnp.int32),
            pltpu.VMEM((_BPW // _PACK, 1, 128), jnp.float32),
            pltpu.SemaphoreType.DMA,
        ],
    )
    def k(tbl_hbm, i_hbm, out_hbm, iv, sel_v, sem):
        wid = lax.axis_index("s") * _NC + lax.axis_index("c")
        base = wid * _BPW
        out_base = wid * (_BPW // _PACK)
        pltpu.sync_copy(i_hbm.at[pl.ds(base, _BPW)], iv)

        @pl.loop(0, _BPW // _G)
        def _grp(g):
            i16 = iv[pl.ds(g * _G, _G)]
            c16 = lax.shift_right_logical(i16, 3)
            q16 = lax.bitwise_and(i16, 7)
            for j in range(_G):
                dr = g * (_G // _PACK) + j // _PACK
                lane = (j % _PACK) * _DIM
                pltpu.async_copy(
                    tbl_hbm.at[c16[j], q16[j], :],
                    sel_v.at[dr, 0, pl.ds(lane, _DIM)],
                    sem,
                )

        pltpu.make_async_copy(
            out_hbm.at[pl.ds(out_base, _BPW // _PACK)], sel_v, sem
        ).wait()
        pltpu.sync_copy(sel_v, out_hbm.at[pl.ds(out_base, _BPW // _PACK)])

    return k(table3, idx)


def _tc_body(ht_ref, relb_ref, gamma_ref, beta_ref, rel_ref,
             preds_ref, loss_ref):
    nrows = _BATCH // _PACK
    h = ht_ref[0:nrows, 0, :]
    t = ht_ref[nrows:2 * nrows, 0, :]

    jj = lax.broadcasted_iota(jnp.int32, (128, 128), 0)
    kk = lax.broadcasted_iota(jnp.int32, (128, 128), 1)
    fold = jnp.where((jj % _DIM) == (kk % _DIM), 1.0 / _BATCH, 0.0)

    def batchnorm(x):
        m = lax.dot(jnp.sum(x, axis=0, keepdims=True), fold,
                    preferred_element_type=jnp.float32)
        xc = x - m
        v = lax.dot(jnp.sum(xc * xc, axis=0, keepdims=True), fold,
                    preferred_element_type=jnp.float32)
        scale = gamma_ref[...] * lax.rsqrt(v + 1e-5)
        return xc * scale + beta_ref[...]

    hn = batchnorm(h)
    tn = batchnorm(t)

    rel_id = relb_ref[...] % _NUM_REL
    r = jnp.zeros((nrows, 128), jnp.float32)
    for q in range(_NUM_REL):
        r = r + jnp.where(rel_id == q, rel_ref[q:q + 1, :], 0.0)

    p = hn * tn * r
    sj = lax.broadcasted_iota(jnp.int32, (128, _PACK), 0)
    sq = lax.broadcasted_iota(jnp.int32, (128, _PACK), 1)
    seg = jnp.where((sj // _DIM) == sq, 1.0, 0.0)
    score = lax.dot(p, seg, preferred_element_type=jnp.float32)
    preds = jax.nn.sigmoid(score)
    preds_ref[...] = preds
    loss = -jnp.sum(jnp.log(preds + 1e-10)) / _BATCH
    loss_ref[...] = jnp.broadcast_to(loss, (1, 1))


def _tc_decode(ht4, relb4, gamma128, beta128, rel_tab):
    nrows = _BATCH // _PACK
    return pl.pallas_call(
        _tc_body,
        out_shape=[
            jax.ShapeDtypeStruct((nrows, _PACK), jnp.float32),
            jax.ShapeDtypeStruct((1, 1), jnp.float32),
        ],
    )(ht4, relb4, gamma128, beta128, rel_tab)


def kernel(pos_edges, W, gamma, beta, rel_emb):
    idx = jnp.concatenate([pos_edges[:, 0], pos_edges[:, 2]]).astype(jnp.int32)
    W3 = W.reshape(_NUM_ENT // 8, 8, _DIM)
    ht4 = _sc_gather_rows(W3, idx)
    relb4 = jnp.broadcast_to(
        pos_edges[:, 1][:, None].astype(jnp.int32), (_BATCH, _DIM)
    ).reshape(_BATCH // _PACK, 128)
    gamma128 = jnp.tile(gamma.reshape(1, _DIM), (1, _PACK))
    beta128 = jnp.tile(beta.reshape(1, _DIM), (1, _PACK))
    rel_tab = jnp.tile(rel_emb, (1, _PACK))
    preds4, loss11 = _tc_decode(ht4, relb4, gamma128, beta128, rel_tab)
    return loss11[0, 0], preds4.reshape(_BATCH)


# speedup vs baseline: 1.6526x; 1.0852x over previous
import dataclasses
import functools

import jax
import jax.numpy as jnp
from jax import lax
from jax.experimental import pallas as pl
from jax.experimental.pallas import tpu as pltpu
from jax.experimental.pallas import tpu_sc as plsc

_NUM_ENT = 1000000
_DIM = 32
_NUM_REL = 5
_BATCH = 16384
_NIDX = 2 * _BATCH
_PACK = 128 // _DIM

_NC = 2
_NS = 16
_NW = _NC * _NS
_BPW = _NIDX // _NW
_G = 16


def _sc_gather_rows(table3, c_idx, q_idx):
    mesh = plsc.VectorSubcoreMesh(core_axis_name="c", subcore_axis_name="s")
    cp = pltpu.CompilerParams()
    if "needs_layout_passes" in pltpu.CompilerParams.__dataclass_fields__:
        cp = dataclasses.replace(cp, needs_layout_passes=False)

    @functools.partial(
        pl.kernel,
        mesh=mesh,
        compiler_params=cp,
        out_type=jax.ShapeDtypeStruct((_NIDX // _PACK, 1, 128), jnp.float32),
        scratch_types=[
            pltpu.VMEM((_BPW,), jnp.int32),
            pltpu.VMEM((_BPW,), jnp.int32),
            pltpu.VMEM((_BPW // _PACK, 1, 128), jnp.float32),
            pltpu.SemaphoreType.DMA,
        ],
    )
    def k(tbl_hbm, c_hbm, q_hbm, out_hbm, cv, qv, sel_v, sem):
        wid = lax.axis_index("s") * _NC + lax.axis_index("c")
        base = wid * _BPW
        out_base = wid * (_BPW // _PACK)
        pltpu.sync_copy(c_hbm.at[pl.ds(base, _BPW)], cv)
        pltpu.sync_copy(q_hbm.at[pl.ds(base, _BPW)], qv)

        @pl.loop(0, _BPW // _G)
        def _grp(g):
            c16 = cv[pl.ds(g * _G, _G)]
            q16 = qv[pl.ds(g * _G, _G)]
            for j in range(_G):
                dr = g * (_G // _PACK) + j // _PACK
                lane = (j % _PACK) * _DIM
                pltpu.async_copy(
                    tbl_hbm.at[c16[j], q16[j], :],
                    sel_v.at[dr, 0, pl.ds(lane, _DIM)],
                    sem,
                )

        pltpu.make_async_copy(
            out_hbm.at[pl.ds(out_base, _BPW // _PACK)], sel_v, sem
        ).wait()
        pltpu.sync_copy(sel_v, out_hbm.at[pl.ds(out_base, _BPW // _PACK)])

    return k(table3, c_idx, q_idx)


def _tc_body(ht_ref, relb_ref, gamma_ref, beta_ref, rel_ref,
             preds_ref, loss_ref):
    nrows = _BATCH // _PACK
    h = ht_ref[0:nrows, :]
    t = ht_ref[nrows:2 * nrows, :]

    jj = lax.broadcasted_iota(jnp.int32, (128, 128), 0)
    kk = lax.broadcasted_iota(jnp.int32, (128, 128), 1)
    fold = jnp.where((jj % _DIM) == (kk % _DIM), 1.0 / _BATCH, 0.0)

    def batchnorm(x):
        m = lax.dot(jnp.sum(x, axis=0, keepdims=True), fold,
                    preferred_element_type=jnp.float32)
        xc = x - m
        v = lax.dot(jnp.sum(xc * xc, axis=0, keepdims=True), fold,
                    preferred_element_type=jnp.float32)
        scale = gamma_ref[...] * lax.rsqrt(v + 1e-5)
        return xc * scale + beta_ref[...]

    hn = batchnorm(h)
    tn = batchnorm(t)

    rel_id = relb_ref[...] % _NUM_REL
    r = jnp.zeros((nrows, 128), jnp.float32)
    for q in range(_NUM_REL):
        r = r + jnp.where(rel_id == q, rel_ref[q:q + 1, :], 0.0)

    p = hn * tn * r
    sj = lax.broadcasted_iota(jnp.int32, (128, _PACK), 0)
    sq = lax.broadcasted_iota(jnp.int32, (128, _PACK), 1)
    seg = jnp.where((sj // _DIM) == sq, 1.0, 0.0)
    score = lax.dot(p, seg, preferred_element_type=jnp.float32)
    preds = jax.nn.sigmoid(score)
    preds_ref[...] = preds
    loss = -jnp.sum(jnp.log(preds + 1e-10)) / _BATCH
    loss_ref[...] = jnp.broadcast_to(loss, (1, 1))


def _tc_decode(ht4, relb4, gamma128, beta128, rel_tab):
    nrows = _BATCH // _PACK
    return pl.pallas_call(
        _tc_body,
        out_shape=[
            jax.ShapeDtypeStruct((nrows, _PACK), jnp.float32),
            jax.ShapeDtypeStruct((1, 1), jnp.float32),
        ],
    )(ht4, relb4, gamma128, beta128, rel_tab)


def kernel(pos_edges, W, gamma, beta, rel_emb):
    idx = jnp.concatenate([pos_edges[:, 0], pos_edges[:, 2]]).astype(jnp.int32)
    c_idx = idx // 8
    q_idx = idx % 8
    W3 = W.reshape(_NUM_ENT // 8, 8, _DIM)
    ht4 = _sc_gather_rows(W3, c_idx, q_idx).reshape(_NIDX // _PACK, 128)
    relb4 = jnp.broadcast_to(
        pos_edges[:, 1][:, None].astype(jnp.int32), (_BATCH, _DIM)
    ).reshape(_BATCH // _PACK, 128)
    gamma128 = jnp.tile(gamma.reshape(1, _DIM), (1, _PACK))
    beta128 = jnp.tile(beta.reshape(1, _DIM), (1, _PACK))
    rel_tab = jnp.tile(rel_emb, (1, _PACK))
    preds4, loss11 = _tc_decode(ht4, relb4, gamma128, beta128, rel_tab)
    return loss11[0, 0], preds4.reshape(_BATCH)
